# baseline (speedup 1.0000x reference)
import functools

import jax
import jax.numpy as jnp
from jax import lax
from jax.experimental import pallas as pl
from jax.experimental.pallas import tpu as pltpu
from jax.experimental.pallas import tpu_sc as plsc

G = 128
K = 2
NB = 4

T_SB = 384
T_TAIL = 64


def _transpose_table(table_t, V, D):
    info = plsc.get_sparse_core_info()
    NC = info.num_cores
    NW = NC * info.num_subcores
    n_full = V // T_SB
    per_w = -(-n_full // NW)
    if per_w % 2:
        per_w += 1
    blk_words = T_SB * D

    mesh = plsc.VectorSubcoreMesh(core_axis_name="c", subcore_axis_name="s")

    @functools.partial(
        pl.kernel,
        mesh=mesh,
        out_type=jax.ShapeDtypeStruct((V * D,), jnp.float32),
        scratch_types=[
            pltpu.VMEM((D, T_SB), jnp.float32),
            pltpu.VMEM((D, T_SB), jnp.float32),
            pltpu.VMEM((blk_words,), jnp.float32),
            pltpu.VMEM((blk_words,), jnp.float32),
            pltpu.VMEM((D, 128), jnp.float32),
            [pltpu.SemaphoreType.DMA] * 2,
            [pltpu.SemaphoreType.DMA] * 2,
        ],
        compiler_params=pltpu.CompilerParams(
            use_tc_tiling_on_sc=True, needs_layout_passes=False
        ),
    )
    def tr(tblt_hbm, tail_hbm, out_hbm, in_v0, in_v1, out_v0, out_v1, tail_v, isems, osems):
        w = lax.axis_index("s") * NC + lax.axis_index("c")
        in_bufs, out_bufs = (in_v0, in_v1), (out_v0, out_v1)
        iota = lax.iota(jnp.int32, 16)
        rots = [lax.rem(iota + r, 16) for r in range(16)]
        obase = [rots[r] * D + iota for r in range(16)]

        def blk(k):
            return w + NW * k

        def start_in(k, b):
            @pl.when(blk(k) < n_full)
            def _():
                pltpu.async_copy(
                    tblt_hbm.at[:, pl.ds(blk(k) * T_SB, T_SB)],
                    in_bufs[b], isems[b],
                )

        def wait_in(k, b):
            @pl.when(blk(k) < n_full)
            def _():
                pltpu.make_async_copy(
                    tblt_hbm.at[:, pl.ds(blk(k) * T_SB, T_SB)],
                    in_bufs[b], isems[b],
                ).wait()

        def start_out(k, b):
            @pl.when(blk(k) < n_full)
            def _():
                pltpu.async_copy(
                    out_bufs[b],
                    out_hbm.at[pl.ds(blk(k) * blk_words, blk_words)],
                    osems[b],
                )

        def wait_out(k, b):
            @pl.when((k >= 0) & (blk(k) < n_full))
            def _():
                pltpu.make_async_copy(
                    out_bufs[b],
                    out_hbm.at[pl.ds(blk(k) * blk_words, blk_words)],
                    osems[b],
                ).wait()

        def compute(in_ref, out_ref, width):
            def d_body(di, carry):
                d0 = di * 16
                rows = iota + d0

                def s_body(si, c2):
                    s0 = si * 16
                    for r in range(16):
                        vec = plsc.load_gather(in_ref, [rows, rots[r] + s0])
                        plsc.store_scatter(
                            out_ref, [obase[r] + (s0 * D + d0)], vec
                        )
                    return c2

                lax.fori_loop(0, width // 16, s_body, 0)
                return carry

            lax.fori_loop(0, D // 16, d_body, 0)

        start_in(0, 0)

        def body(i, carry):
            for b in range(2):
                k = i * 2 + b
                start_in(k + 1, 1 - b)
                wait_in(k, b)
                wait_out(k - 2, b)
                compute(in_bufs[b], out_bufs[b], T_SB)
                start_out(k, b)
            return carry

        lax.fori_loop(0, per_w // 2, body, 0)
        for b in range(2):
            wait_out(per_w - 2 + b, b)

        @pl.when(w == 0)
        def _():
            pltpu.sync_copy(tail_hbm, tail_v)
            compute(tail_v, out_v0, 128)
            pltpu.sync_copy(
                out_v0.at[pl.ds(0, 128 * D)],
                out_hbm.at[pl.ds((V - 128) * D, 128 * D)],
            )

    return tr(table_t, table_t[:, V - 128:])


def kernel(x, table):
    B0, B1 = x.shape
    V, D = table.shape
    B = B0 * B1

    info = plsc.get_sparse_core_info()
    NC = info.num_cores
    NW = NC * info.num_subcores
    groups_total = B // G
    g_per_w = groups_total // NW
    n_chunks = g_per_w // K
    assert groups_total % NW == 0 and g_per_w % K == 0 and n_chunks % NB == 0

    idx2d = x.reshape(groups_total, G).astype(jnp.int32)
    tbl_dense = _transpose_table(table.T, V, D).reshape(V, D)

    mesh = plsc.VectorSubcoreMesh(core_axis_name="c", subcore_axis_name="s")

    @functools.partial(
        pl.kernel,
        mesh=mesh,
        out_type=jax.ShapeDtypeStruct((groups_total, G, 2 * D), jnp.float32),
        scratch_types=[
            pltpu.VMEM((NB, K, G), jnp.int32),
            pltpu.VMEM((NB, K, G, D), jnp.float32),
            [pltpu.SemaphoreType.DMA] * NB,
            [pltpu.SemaphoreType.DMA] * NB,
        ],
        compiler_params=pltpu.CompilerParams(use_tc_tiling_on_sc=False),
    )
    def emb(idx_hbm, table_hbm, out_hbm, idx_v, rows_v, gsems, osems):
        wid = lax.axis_index("s") * NC + lax.axis_index("c")
        g0 = wid * g_per_w

        def fire(c, b):
            pltpu.sync_copy(idx_hbm.at[pl.ds(g0 + c * K, K)], idx_v.at[b])
            for j in range(K):
                pltpu.async_copy(
                    table_hbm.at[idx_v.at[b, j]], rows_v.at[b, j], gsems[b]
                )

        def wait_gathers(b):
            for j in range(K):
                pltpu.make_async_copy(
                    table_hbm.at[idx_v.at[b, j]], rows_v.at[b, j], gsems[b]
                ).wait()

        def put_out(c, b):
            pltpu.async_copy(
                rows_v.at[b],
                out_hbm.at[pl.ds(g0 + c * K, K), slice(None), pl.ds(0, D)],
                osems[b],
            )

        def wait_out(c, b):
            pltpu.make_async_copy(
                rows_v.at[b],
                out_hbm.at[pl.ds(g0 + c * K, K), slice(None), pl.ds(0, D)],
                osems[b],
            ).wait()

        for b in range(NB):
            fire(b, b)

        def body(cc, carry):
            for b in range(NB):
                c = cc + b
                wait_gathers(b)
                put_out(c, b)
                wait_out(c, b)
                fire(c + NB, b)
            return carry

        lax.fori_loop(0, n_chunks // NB - 1, lambda i, car: body(i * NB, car), 0)

        for b in range(NB):
            c = n_chunks - NB + b
            wait_gathers(b)
            put_out(c, b)
            wait_out(c, b)

    out = emb(idx2d, tbl_dense)
    return out.reshape(B0, B1, 2 * D)[:, :, :D]


# speedup vs baseline: 1.0153x; 1.0153x over previous
import functools

import jax
import jax.numpy as jnp
from jax import lax
from jax.experimental import pallas as pl
from jax.experimental.pallas import tpu as pltpu
from jax.experimental.pallas import tpu_sc as plsc

G = 128
K = 2
NB = 4

T_SB = 128
NBT = 4
T_TAIL = 64


def _transpose_table(table_t, V, D):
    info = plsc.get_sparse_core_info()
    NC = info.num_cores
    NW = NC * info.num_subcores
    n_full = V // T_SB
    per_w = -(-n_full // NW)
    per_w = -(-per_w // NBT) * NBT
    blk_words = T_SB * D

    mesh = plsc.VectorSubcoreMesh(core_axis_name="c", subcore_axis_name="s")

    @functools.partial(
        pl.kernel,
        mesh=mesh,
        out_type=jax.ShapeDtypeStruct((V * D,), jnp.float32),
        scratch_types=[
            [pltpu.VMEM((D, T_SB), jnp.float32)] * NBT,
            [pltpu.VMEM((blk_words,), jnp.float32)] * NBT,
            pltpu.VMEM((D, 128), jnp.float32),
            [pltpu.SemaphoreType.DMA] * NBT,
            [pltpu.SemaphoreType.DMA] * NBT,
        ],
        compiler_params=pltpu.CompilerParams(
            use_tc_tiling_on_sc=True, needs_layout_passes=False
        ),
    )
    def tr(tblt_hbm, tail_hbm, out_hbm, in_bufs, out_bufs, tail_v, isems, osems):
        w = lax.axis_index("s") * NC + lax.axis_index("c")
        iota = lax.iota(jnp.int32, 16)
        rots = [lax.rem(iota + r, 16) for r in range(16)]
        obase = [rots[r] * D + iota for r in range(16)]

        def blk(k):
            return w + NW * k

        def start_in(k, b):
            @pl.when(blk(k) < n_full)
            def _():
                pltpu.async_copy(
                    tblt_hbm.at[:, pl.ds(blk(k) * T_SB, T_SB)],
                    in_bufs[b], isems[b],
                )

        def wait_in(k, b):
            @pl.when(blk(k) < n_full)
            def _():
                pltpu.make_async_copy(
                    tblt_hbm.at[:, pl.ds(blk(k) * T_SB, T_SB)],
                    in_bufs[b], isems[b],
                ).wait()

        def start_out(k, b):
            @pl.when(blk(k) < n_full)
            def _():
                pltpu.async_copy(
                    out_bufs[b],
                    out_hbm.at[pl.ds(blk(k) * blk_words, blk_words)],
                    osems[b],
                )

        def wait_out(k, b):
            @pl.when((k >= 0) & (blk(k) < n_full))
            def _():
                pltpu.make_async_copy(
                    out_bufs[b],
                    out_hbm.at[pl.ds(blk(k) * blk_words, blk_words)],
                    osems[b],
                ).wait()

        def compute(in_ref, out_ref, width):
            def d_body(di, carry):
                d0 = di * 16
                rows = iota + d0

                def s_body(si, c2):
                    s0 = si * 16
                    for r in range(16):
                        vec = plsc.load_gather(in_ref, [rows, rots[r] + s0])
                        plsc.store_scatter(
                            out_ref, [obase[r] + (s0 * D + d0)], vec
                        )
                    return c2

                lax.fori_loop(0, width // 16, s_body, 0)
                return carry

            lax.fori_loop(0, D // 16, d_body, 0)

        for b in range(NBT):
            start_in(b, b)

        def body(i, carry):
            for b in range(NBT):
                k = i * NBT + b
                wait_in(k, b)
                wait_out(k - NBT, b)
                compute(in_bufs[b], out_bufs[b], T_SB)
                start_out(k, b)
                start_in(k + NBT, b)
            return carry

        lax.fori_loop(0, per_w // NBT, body, 0)
        for b in range(NBT):
            wait_out(per_w - NBT + b, b)

        @pl.when(w == 0)
        def _():
            pltpu.sync_copy(tail_hbm, tail_v)
            compute(tail_v, out_bufs[0], 128)
            pltpu.sync_copy(
                out_bufs[0].at[pl.ds(0, 128 * D)],
                out_hbm.at[pl.ds((V - 128) * D, 128 * D)],
            )

    return tr(table_t, table_t[:, V - 128:])


def kernel(x, table):
    B0, B1 = x.shape
    V, D = table.shape
    B = B0 * B1

    info = plsc.get_sparse_core_info()
    NC = info.num_cores
    NW = NC * info.num_subcores
    groups_total = B // G
    g_per_w = groups_total // NW
    n_chunks = g_per_w // K
    assert groups_total % NW == 0 and g_per_w % K == 0 and n_chunks % NB == 0

    idx2d = x.reshape(groups_total, G).astype(jnp.int32)
    tbl_dense = _transpose_table(table.T, V, D).reshape(V, D)

    mesh = plsc.VectorSubcoreMesh(core_axis_name="c", subcore_axis_name="s")

    @functools.partial(
        pl.kernel,
        mesh=mesh,
        out_type=jax.ShapeDtypeStruct((groups_total, G, 2 * D), jnp.float32),
        scratch_types=[
            pltpu.VMEM((NB, K, G), jnp.int32),
            pltpu.VMEM((NB, K, G, D), jnp.float32),
            [pltpu.SemaphoreType.DMA] * NB,
            [pltpu.SemaphoreType.DMA] * NB,
        ],
        compiler_params=pltpu.CompilerParams(use_tc_tiling_on_sc=False),
    )
    def emb(idx_hbm, table_hbm, out_hbm, idx_v, rows_v, gsems, osems):
        wid = lax.axis_index("s") * NC + lax.axis_index("c")
        g0 = wid * g_per_w

        def fire(c, b):
            pltpu.sync_copy(idx_hbm.at[pl.ds(g0 + c * K, K)], idx_v.at[b])
            for j in range(K):
                pltpu.async_copy(
                    table_hbm.at[idx_v.at[b, j]], rows_v.at[b, j], gsems[b]
                )

        def wait_gathers(b):
            for j in range(K):
                pltpu.make_async_copy(
                    table_hbm.at[idx_v.at[b, j]], rows_v.at[b, j], gsems[b]
                ).wait()

        def put_out(c, b):
            pltpu.async_copy(
                rows_v.at[b],
                out_hbm.at[pl.ds(g0 + c * K, K), slice(None), pl.ds(0, D)],
                osems[b],
            )

        def wait_out(c, b):
            pltpu.make_async_copy(
                rows_v.at[b],
                out_hbm.at[pl.ds(g0 + c * K, K), slice(None), pl.ds(0, D)],
                osems[b],
            ).wait()

        for b in range(NB):
            fire(b, b)

        def body(cc, carry):
            for b in range(NB):
                c = cc + b
                wait_gathers(b)
                put_out(c, b)
                wait_out(c, b)
                fire(c + NB, b)
            return carry

        lax.fori_loop(0, n_chunks // NB - 1, lambda i, car: body(i * NB, car), 0)

        for b in range(NB):
            c = n_chunks - NB + b
            wait_gathers(b)
            put_out(c, b)
            wait_out(c, b)

    out = emb(idx2d, tbl_dense)
    return out.reshape(B0, B1, 2 * D)[:, :, :D]


# speedup vs baseline: 1.5071x; 1.4844x over previous
import functools

import jax
import jax.numpy as jnp
from jax import lax
from jax.experimental import pallas as pl
from jax.experimental.pallas import tpu as pltpu
from jax.experimental.pallas import tpu_sc as plsc

G = 128
K = 2
NB = 4

T_SB = 128
NBT = 4
T_TAIL = 64


def _transpose_table(table_t, V, D):
    info = plsc.get_sparse_core_info()
    NC = info.num_cores
    NW = NC * info.num_subcores
    n_full = V // T_SB
    per_w = -(-n_full // NW)
    per_w = -(-per_w // NBT) * NBT
    blk_words = T_SB * D

    mesh = plsc.VectorSubcoreMesh(core_axis_name="c", subcore_axis_name="s")

    @functools.partial(
        pl.kernel,
        mesh=mesh,
        out_type=jax.ShapeDtypeStruct((V * D,), jnp.float32),
        scratch_types=[
            [pltpu.VMEM((D, T_SB), jnp.float32)] * NBT,
            [pltpu.VMEM((blk_words,), jnp.float32)] * NBT,
            pltpu.VMEM((D, 128), jnp.float32),
            [pltpu.SemaphoreType.DMA] * NBT,
            [pltpu.SemaphoreType.DMA] * NBT,
        ],
        compiler_params=pltpu.CompilerParams(
            use_tc_tiling_on_sc=True, needs_layout_passes=False
        ),
    )
    def tr(tblt_hbm, tail_hbm, out_hbm, in_bufs, out_bufs, tail_v, isems, osems):
        w = lax.axis_index("s") * NC + lax.axis_index("c")
        iota = lax.iota(jnp.int32, 16)
        rots = [lax.rem(iota + r, 16) for r in range(16)]
        obase = [rots[r] * D + iota for r in range(16)]

        def blk(k):
            return w + NW * k

        def start_in(k, b):
            @pl.when(blk(k) < n_full)
            def _():
                pltpu.async_copy(
                    tblt_hbm.at[:, pl.ds(blk(k) * T_SB, T_SB)],
                    in_bufs[b], isems[b],
                )

        def wait_in(k, b):
            @pl.when(blk(k) < n_full)
            def _():
                pltpu.make_async_copy(
                    tblt_hbm.at[:, pl.ds(blk(k) * T_SB, T_SB)],
                    in_bufs[b], isems[b],
                ).wait()

        def start_out(k, b):
            @pl.when(blk(k) < n_full)
            def _():
                pltpu.async_copy(
                    out_bufs[b],
                    out_hbm.at[pl.ds(blk(k) * blk_words, blk_words)],
                    osems[b],
                )

        def wait_out(k, b):
            @pl.when((k >= 0) & (blk(k) < n_full))
            def _():
                pltpu.make_async_copy(
                    out_bufs[b],
                    out_hbm.at[pl.ds(blk(k) * blk_words, blk_words)],
                    osems[b],
                ).wait()

        def compute(in_ref, out_ref, width):
            def s_body(si, carry):
                s0 = si * 16
                cols = [rots[r] + s0 for r in range(16)]
                obs = [obase[r] + s0 * D for r in range(16)]

                def d_body(di, c2):
                    d0 = di * 16
                    rows = iota + d0
                    vecs = [
                        plsc.load_gather(in_ref, [rows, cols[r]])
                        for r in range(16)
                    ]
                    for r in range(16):
                        plsc.store_scatter(out_ref, [obs[r] + d0], vecs[r])
                    return c2

                lax.fori_loop(0, D // 16, d_body, 0)
                return carry

            lax.fori_loop(0, width // 16, s_body, 0)

        for b in range(NBT):
            start_in(b, b)

        def body(i, carry):
            for b in range(NBT):
                k = i * NBT + b
                wait_in(k, b)
                wait_out(k - NBT, b)
                compute(in_bufs[b], out_bufs[b], T_SB)
                start_out(k, b)
                start_in(k + NBT, b)
            return carry

        lax.fori_loop(0, per_w // NBT, body, 0)
        for b in range(NBT):
            wait_out(per_w - NBT + b, b)

        @pl.when(w == 0)
        def _():
            pltpu.sync_copy(tail_hbm, tail_v)
            compute(tail_v, out_bufs[0], 128)
            pltpu.sync_copy(
                out_bufs[0].at[pl.ds(0, 128 * D)],
                out_hbm.at[pl.ds((V - 128) * D, 128 * D)],
            )

    return tr(table_t, table_t[:, V - 128:])


def kernel(x, table):
    B0, B1 = x.shape
    V, D = table.shape
    B = B0 * B1

    info = plsc.get_sparse_core_info()
    NC = info.num_cores
    NW = NC * info.num_subcores
    groups_total = B // G
    g_per_w = groups_total // NW
    n_chunks = g_per_w // K
    assert groups_total % NW == 0 and g_per_w % K == 0 and n_chunks % NB == 0

    idx2d = x.reshape(groups_total, G).astype(jnp.int32)
    tbl_dense = _transpose_table(table.T, V, D).reshape(V, D)

    mesh = plsc.VectorSubcoreMesh(core_axis_name="c", subcore_axis_name="s")

    @functools.partial(
        pl.kernel,
        mesh=mesh,
        out_type=jax.ShapeDtypeStruct((groups_total, G, 2 * D), jnp.float32),
        scratch_types=[
            pltpu.VMEM((NB, K, G), jnp.int32),
            pltpu.VMEM((NB, K, G, D), jnp.float32),
            [pltpu.SemaphoreType.DMA] * NB,
            [pltpu.SemaphoreType.DMA] * NB,
        ],
        compiler_params=pltpu.CompilerParams(use_tc_tiling_on_sc=False),
    )
    def emb(idx_hbm, table_hbm, out_hbm, idx_v, rows_v, gsems, osems):
        wid = lax.axis_index("s") * NC + lax.axis_index("c")
        g0 = wid * g_per_w

        def fire(c, b):
            pltpu.sync_copy(idx_hbm.at[pl.ds(g0 + c * K, K)], idx_v.at[b])
            for j in range(K):
                pltpu.async_copy(
                    table_hbm.at[idx_v.at[b, j]], rows_v.at[b, j], gsems[b]
                )

        def wait_gathers(b):
            for j in range(K):
                pltpu.make_async_copy(
                    table_hbm.at[idx_v.at[b, j]], rows_v.at[b, j], gsems[b]
                ).wait()

        def put_out(c, b):
            pltpu.async_copy(
                rows_v.at[b],
                out_hbm.at[pl.ds(g0 + c * K, K), slice(None), pl.ds(0, D)],
                osems[b],
            )

        def wait_out(c, b):
            pltpu.make_async_copy(
                rows_v.at[b],
                out_hbm.at[pl.ds(g0 + c * K, K), slice(None), pl.ds(0, D)],
                osems[b],
            ).wait()

        for b in range(NB):
            fire(b, b)

        def body(cc, carry):
            for b in range(NB):
                c = cc + b
                wait_gathers(b)
                put_out(c, b)
                wait_out(c, b)
                fire(c + NB, b)
            return carry

        lax.fori_loop(0, n_chunks // NB - 1, lambda i, car: body(i * NB, car), 0)

        for b in range(NB):
            c = n_chunks - NB + b
            wait_gathers(b)
            put_out(c, b)
            wait_out(c, b)

    out = emb(idx2d, tbl_dense)
    return out.reshape(B0, B1, 2 * D)[:, :, :D]
